# baseline (speedup 1.0000x reference)
import functools

import jax
import jax.numpy as jnp
from jax import lax
from jax.experimental import pallas as pl
from jax.experimental.pallas import tpu as pltpu
from jax.experimental.pallas import tpu_sc as plsc

_B, _L, _V, _D = 16384, 26, 1000000, 32
_BT = _B * _L


def _mesh_info():
    info = plsc.get_sparse_core_info()
    return info.num_cores, info.num_subcores


def _make_k1():
    NC, NS = _mesh_info()
    NW = NC * NS
    R = 512
    NCH = 1953
    TAIL0 = NCH * R
    TAILR = _V - TAIL0
    PC = NCH // NW
    mesh = plsc.VectorSubcoreMesh(core_axis_name="c", subcore_axis_name="s")

    @functools.partial(
        pl.kernel,
        mesh=mesh,
        out_type=jax.ShapeDtypeStruct((_V // 4, 128), jnp.float32),
        scratch_types=[
            pltpu.VMEM((2, _D, R), jnp.float32),
            pltpu.VMEM((2, R // 4, 128), jnp.float32),
            pltpu.VMEM((_D, TAILR), jnp.float32),
            pltpu.VMEM((TAILR // 4, 128), jnp.float32),
            pltpu.SemaphoreType.DMA,
            pltpu.SemaphoreType.DMA,
            pltpu.SemaphoreType.DMA,
            pltpu.SemaphoreType.DMA,
        ],
        compiler_params=pltpu.CompilerParams(
            use_tc_tiling_on_sc=True,
            needs_layout_passes=False,
            disable_bounds_checks=True,
        ),
    )
    def k1(tabT_hbm, tail_hbm, t128_hbm, in_v, out_v, tail_v, tailo_v,
           isem0, isem1, osem0, osem1):
        wid = lax.axis_index("s") * NC + lax.axis_index("c")
        base = wid * PC
        lane = lax.iota(jnp.int32, 16)
        rpat = lax.shift_right_logical(lane, 2)
        cpat = lax.shift_left(lane & 3, 5)
        isems = (isem0, isem1)
        osems = (osem0, osem1)

        def start_in(c, b):
            r0 = pl.multiple_of(c * R, R)
            pltpu.async_copy(tabT_hbm.at[:, pl.ds(r0, R)], in_v.at[b],
                             isems[b])

        def wait_in(b):
            pltpu.make_async_copy(tabT_hbm.at[:, pl.ds(0, R)], in_v.at[b],
                                  isems[b]).wait()

        def start_out(c, b):
            q0 = pl.multiple_of(c * (R // 4), R // 4)
            pltpu.async_copy(out_v.at[b], t128_hbm.at[pl.ds(q0, R // 4)],
                             osems[b])

        def wait_out(b):
            pltpu.make_async_copy(out_v.at[b], t128_hbm.at[pl.ds(0, R // 4)],
                                  osems[b]).wait()

        def transpose(ib, ob):
            def floop(f, _):
                cv = cpat + f
                def tloop(t, rv):
                    vals = ib[f, pl.ds(16 * t, 16)]
                    plsc.store_scatter(ob, [rv, cv], vals)
                    return rv + 4
                lax.fori_loop(0, R // 16, tloop, rpat, unroll=8)
                return ()
            lax.fori_loop(0, _D, floop, ())

        start_in(base + 0, 0)
        start_in(base + 1, 1)

        def pair(j, _):
            c0 = base + 2 * j
            wait_in(0)
            @pl.when(j > 0)
            def _():
                wait_out(0)
            transpose(in_v.at[0], out_v.at[0])
            start_in(c0 + 2, 0)
            start_out(c0, 0)
            wait_in(1)
            @pl.when(j > 0)
            def _():
                wait_out(1)
            transpose(in_v.at[1], out_v.at[1])
            @pl.when(2 * j + 3 < PC)
            def _():
                start_in(c0 + 3, 1)
            start_out(c0 + 1, 1)
            return ()

        lax.fori_loop(0, PC // 2, pair, ())

        wait_in(0)
        wait_out(0)
        transpose(in_v.at[0], out_v.at[0])
        start_out(base + PC - 1, 0)
        wait_out(0)
        wait_out(1)

        @pl.when(wid == NW - 1)
        def _():
            r0 = pl.multiple_of((NCH - 1) * R, R)
            pltpu.sync_copy(tabT_hbm.at[:, pl.ds(r0, R)], in_v.at[0])
            transpose(in_v.at[0], out_v.at[0])
            q0 = pl.multiple_of((NCH - 1) * (R // 4), R // 4)
            pltpu.sync_copy(out_v.at[0], t128_hbm.at[pl.ds(q0, R // 4)])

            pltpu.sync_copy(tail_hbm, tail_v)
            def row(dq, _):
                for v in range(8):
                    rowidx = lane + (16 * (v % 2))
                    colidx = jnp.broadcast_to(4 * dq + (v // 2), (16,))
                    vals = plsc.load_gather(tail_v, [rowidx, colidx])
                    tailo_v[dq, pl.ds(16 * v, 16)] = vals
                return ()
            lax.fori_loop(0, TAILR // 4, row, ())
            tq = pl.multiple_of(TAIL0 // 4, 8)
            pltpu.sync_copy(tailo_v, t128_hbm.at[pl.ds(tq, TAILR // 4)])

    return k1


def _make_k2():
    NC, NS = _mesh_info()
    NW = NC * NS
    b_per_w = _BT // NW
    NCHUNK = 8
    C = b_per_w // NCHUNK
    mesh = plsc.VectorSubcoreMesh(core_axis_name="c", subcore_axis_name="s")

    @functools.partial(
        pl.kernel,
        mesh=mesh,
        out_type=jax.ShapeDtypeStruct((_BT, _D), jnp.float32),
        scratch_types=[
            pltpu.VMEM((2, C), jnp.int32),
            pltpu.VMEM((2, C, _D), jnp.float32),
            pltpu.SemaphoreType.DMA,
            pltpu.SemaphoreType.DMA,
        ],
        compiler_params=pltpu.CompilerParams(use_tc_tiling_on_sc=False),
    )
    def k2(idx_hbm, table_hbm, out_hbm, idx_v, rows_v, gsem, osem):
        wid = lax.axis_index("s") * NC + lax.axis_index("c")
        base = wid * b_per_w
        gathers = [None, None]
        stores = [None, None]
        for c in range(2):
            b = c % 2
            pltpu.sync_copy(idx_hbm.at[pl.ds(base + c * C, C)], idx_v.at[b])
            gathers[b] = pltpu.async_copy(
                table_hbm.at[idx_v.at[b]], rows_v.at[b], gsem
            )
        for c in range(NCHUNK):
            b = c % 2
            gathers[b].wait()
            stores[b] = pltpu.async_copy(
                rows_v.at[b], out_hbm.at[pl.ds(base + c * C, C)], osem
            )
            if c + 2 < NCHUNK:
                pltpu.sync_copy(
                    idx_hbm.at[pl.ds(base + (c + 2) * C, C)], idx_v.at[b]
                )
                stores[b].wait()
                gathers[b] = pltpu.async_copy(
                    table_hbm.at[idx_v.at[b]], rows_v.at[b], gsem
                )
        stores[(NCHUNK - 2) % 2].wait()
        stores[(NCHUNK - 1) % 2].wait()

    return k2


def _make_k3():
    NC, NS = _mesh_info()
    NW = NC * NS
    CB = 512
    NCH = _L * (_B // CB)
    per_w = NCH // NW
    mesh = plsc.VectorSubcoreMesh(core_axis_name="c", subcore_axis_name="s")

    @functools.partial(
        pl.kernel,
        mesh=mesh,
        out_type=jax.ShapeDtypeStruct((_L, _D, _B), jnp.float32),
        scratch_types=[
            pltpu.VMEM((2, CB // 4, 128), jnp.float32),
            pltpu.VMEM((2, _D, CB), jnp.float32),
            pltpu.SemaphoreType.DMA,
            pltpu.SemaphoreType.DMA,
            pltpu.SemaphoreType.DMA,
            pltpu.SemaphoreType.DMA,
        ],
        compiler_params=pltpu.CompilerParams(
            use_tc_tiling_on_sc=True,
            needs_layout_passes=False,
            disable_bounds_checks=True,
        ),
    )
    def k3(g_hbm, out_hbm, in_v, out_v, isem0, isem1, osem0, osem1):
        wid = lax.axis_index("s") * NC + lax.axis_index("c")
        base = wid * per_w
        lane = lax.iota(jnp.int32, 16)
        rpat = lax.shift_right_logical(lane, 2)
        cpat = lax.shift_left(lane & 3, 5)
        isems = (isem0, isem1)
        osems = (osem0, osem1)

        def locate(c):
            l = c // (_B // CB)
            blk = c % (_B // CB)
            b0 = pl.multiple_of(blk * CB, CB)
            g0 = pl.multiple_of(l * (_B // 4) + blk * (CB // 4), CB // 4)
            return l, b0, g0

        def start_in(c, b):
            _, _, g0 = locate(c)
            pltpu.async_copy(g_hbm.at[pl.ds(g0, CB // 4)], in_v.at[b],
                             isems[b])

        def wait_in(b):
            pltpu.make_async_copy(g_hbm.at[pl.ds(0, CB // 4)], in_v.at[b],
                                  isems[b]).wait()

        def start_out(c, b):
            l, b0, _ = locate(c)
            pltpu.async_copy(out_v.at[b], out_hbm.at[l, :, pl.ds(b0, CB)],
                             osems[b])

        def wait_out(b):
            pltpu.make_async_copy(out_v.at[b], out_hbm.at[0, :, pl.ds(0, CB)],
                                  osems[b]).wait()

        def transpose(ib, ob):
            def floop(f, _):
                cv = cpat + f
                def kloop(k, rv):
                    vals = plsc.load_gather(ib, [rv, cv])
                    ob[f, pl.ds(16 * k, 16)] = vals
                    return rv + 4
                lax.fori_loop(0, CB // 16, kloop, rpat, unroll=8)
                return ()
            lax.fori_loop(0, _D, floop, ())

        start_in(base + 0, 0)
        start_in(base + 1, 1)

        def pair(j, _):
            c0 = base + 2 * j
            wait_in(0)
            @pl.when(j > 0)
            def _():
                wait_out(0)
            transpose(in_v.at[0], out_v.at[0])
            @pl.when(2 * j + 2 < per_w)
            def _():
                start_in(c0 + 2, 0)
            start_out(c0, 0)
            wait_in(1)
            @pl.when(j > 0)
            def _():
                wait_out(1)
            transpose(in_v.at[1], out_v.at[1])
            @pl.when(2 * j + 3 < per_w)
            def _():
                start_in(c0 + 3, 1)
            start_out(c0 + 1, 1)
            return ()

        lax.fori_loop(0, per_w // 2, pair, ())
        wait_out(0)
        wait_out(1)

    return k3


def kernel(indices, table):
    tabT = table.T
    tail = lax.slice(tabT, (0, 999936), (_D, _V))
    t128 = _make_k1()(tabT, tail)
    tbl = jnp.reshape(t128, (_V, _D))
    flat = jnp.reshape(jnp.transpose(indices), (_BT,))
    g = _make_k2()(flat, tbl)
    g128 = jnp.reshape(g, (_BT // 4, 128))
    o = _make_k3()(g128)
    return jnp.transpose(o, (2, 0, 1))


# speedup vs baseline: 1.0796x; 1.0796x over previous
import functools

import jax
import jax.numpy as jnp
from jax import lax
from jax.experimental import pallas as pl
from jax.experimental.pallas import tpu as pltpu
from jax.experimental.pallas import tpu_sc as plsc

_B, _L, _V, _D = 16384, 26, 1000000, 32
_BT = _B * _L


def _mesh_info():
    info = plsc.get_sparse_core_info()
    return info.num_cores, info.num_subcores


def _make_k1():
    NC, NS = _mesh_info()
    NW = NC * NS
    R = 512
    NCH = 1953
    TAIL0 = NCH * R
    TAILR = _V - TAIL0
    PC = NCH // NW
    mesh = plsc.VectorSubcoreMesh(core_axis_name="c", subcore_axis_name="s")

    @functools.partial(
        pl.kernel,
        mesh=mesh,
        out_type=jax.ShapeDtypeStruct((_V // 4, 128), jnp.float32),
        scratch_types=[
            pltpu.VMEM((2, _D, R), jnp.float32),
            pltpu.VMEM((2, R // 4, 128), jnp.float32),
            pltpu.VMEM((_D, TAILR), jnp.float32),
            pltpu.VMEM((TAILR // 4, 128), jnp.float32),
            pltpu.SemaphoreType.DMA,
            pltpu.SemaphoreType.DMA,
            pltpu.SemaphoreType.DMA,
            pltpu.SemaphoreType.DMA,
        ],
        compiler_params=pltpu.CompilerParams(
            use_tc_tiling_on_sc=True,
            needs_layout_passes=False,
            disable_bounds_checks=True,
        ),
    )
    def k1(tabT_hbm, tail_hbm, t128_hbm, in_v, out_v, tail_v, tailo_v,
           isem0, isem1, osem0, osem1):
        wid = lax.axis_index("s") * NC + lax.axis_index("c")
        base = wid * PC
        lane = lax.iota(jnp.int32, 16)
        rpat = lax.shift_right_logical(lane, 2)
        cpat = lax.shift_left(lane & 3, 5)
        isems = (isem0, isem1)
        osems = (osem0, osem1)

        def start_in(c, b):
            r0 = pl.multiple_of(c * R, R)
            pltpu.async_copy(tabT_hbm.at[:, pl.ds(r0, R)], in_v.at[b],
                             isems[b])

        def wait_in(b):
            pltpu.make_async_copy(tabT_hbm.at[:, pl.ds(0, R)], in_v.at[b],
                                  isems[b]).wait()

        def start_out(c, b):
            q0 = pl.multiple_of(c * (R // 4), R // 4)
            pltpu.async_copy(out_v.at[b], t128_hbm.at[pl.ds(q0, R // 4)],
                             osems[b])

        def wait_out(b):
            pltpu.make_async_copy(out_v.at[b], t128_hbm.at[pl.ds(0, R // 4)],
                                  osems[b]).wait()

        def transpose(ib, ob):
            def floop(f, _):
                cv = cpat + f
                @plsc.parallel_loop(0, R // 16, unroll=8)
                def _(t):
                    vals = ib[f, pl.ds(16 * t, 16)]
                    plsc.store_scatter(ob, [rpat + 4 * t, cv], vals)
                return ()
            lax.fori_loop(0, _D, floop, ())

        start_in(base + 0, 0)
        start_in(base + 1, 1)

        def pair(j, _):
            c0 = base + 2 * j
            wait_in(0)
            @pl.when(j > 0)
            def _():
                wait_out(0)
            transpose(in_v.at[0], out_v.at[0])
            start_in(c0 + 2, 0)
            start_out(c0, 0)
            wait_in(1)
            @pl.when(j > 0)
            def _():
                wait_out(1)
            transpose(in_v.at[1], out_v.at[1])
            @pl.when(2 * j + 3 < PC)
            def _():
                start_in(c0 + 3, 1)
            start_out(c0 + 1, 1)
            return ()

        lax.fori_loop(0, PC // 2, pair, ())

        wait_in(0)
        wait_out(0)
        transpose(in_v.at[0], out_v.at[0])
        start_out(base + PC - 1, 0)
        wait_out(0)
        wait_out(1)

        @pl.when(wid == NW - 1)
        def _():
            r0 = pl.multiple_of((NCH - 1) * R, R)
            pltpu.sync_copy(tabT_hbm.at[:, pl.ds(r0, R)], in_v.at[0])
            transpose(in_v.at[0], out_v.at[0])
            q0 = pl.multiple_of((NCH - 1) * (R // 4), R // 4)
            pltpu.sync_copy(out_v.at[0], t128_hbm.at[pl.ds(q0, R // 4)])

            pltpu.sync_copy(tail_hbm, tail_v)
            def row(dq, _):
                for v in range(8):
                    rowidx = lane + (16 * (v % 2))
                    colidx = jnp.broadcast_to(4 * dq + (v // 2), (16,))
                    vals = plsc.load_gather(tail_v, [rowidx, colidx])
                    tailo_v[dq, pl.ds(16 * v, 16)] = vals
                return ()
            lax.fori_loop(0, TAILR // 4, row, ())
            tq = pl.multiple_of(TAIL0 // 4, 8)
            pltpu.sync_copy(tailo_v, t128_hbm.at[pl.ds(tq, TAILR // 4)])

    return k1


def _make_k2():
    NC, NS = _mesh_info()
    NW = NC * NS
    b_per_w = _BT // NW
    NCHUNK = 8
    C = b_per_w // NCHUNK
    mesh = plsc.VectorSubcoreMesh(core_axis_name="c", subcore_axis_name="s")

    @functools.partial(
        pl.kernel,
        mesh=mesh,
        out_type=jax.ShapeDtypeStruct((_BT, _D), jnp.float32),
        scratch_types=[
            pltpu.VMEM((2, C), jnp.int32),
            pltpu.VMEM((2, C, _D), jnp.float32),
            pltpu.SemaphoreType.DMA,
            pltpu.SemaphoreType.DMA,
        ],
        compiler_params=pltpu.CompilerParams(use_tc_tiling_on_sc=False),
    )
    def k2(idx_hbm, table_hbm, out_hbm, idx_v, rows_v, gsem, osem):
        wid = lax.axis_index("s") * NC + lax.axis_index("c")
        base = wid * b_per_w
        gathers = [None, None]
        stores = [None, None]
        for c in range(2):
            b = c % 2
            pltpu.sync_copy(idx_hbm.at[pl.ds(base + c * C, C)], idx_v.at[b])
            gathers[b] = pltpu.async_copy(
                table_hbm.at[idx_v.at[b]], rows_v.at[b], gsem
            )
        for c in range(NCHUNK):
            b = c % 2
            gathers[b].wait()
            stores[b] = pltpu.async_copy(
                rows_v.at[b], out_hbm.at[pl.ds(base + c * C, C)], osem
            )
            if c + 2 < NCHUNK:
                pltpu.sync_copy(
                    idx_hbm.at[pl.ds(base + (c + 2) * C, C)], idx_v.at[b]
                )
                stores[b].wait()
                gathers[b] = pltpu.async_copy(
                    table_hbm.at[idx_v.at[b]], rows_v.at[b], gsem
                )
        stores[(NCHUNK - 2) % 2].wait()
        stores[(NCHUNK - 1) % 2].wait()

    return k2


def _make_k3():
    NC, NS = _mesh_info()
    NW = NC * NS
    CB = 512
    NCH = _L * (_B // CB)
    per_w = NCH // NW
    mesh = plsc.VectorSubcoreMesh(core_axis_name="c", subcore_axis_name="s")

    @functools.partial(
        pl.kernel,
        mesh=mesh,
        out_type=jax.ShapeDtypeStruct((_L, _D, _B), jnp.float32),
        scratch_types=[
            pltpu.VMEM((2, CB // 4, 128), jnp.float32),
            pltpu.VMEM((2, _D, CB), jnp.float32),
            pltpu.SemaphoreType.DMA,
            pltpu.SemaphoreType.DMA,
            pltpu.SemaphoreType.DMA,
            pltpu.SemaphoreType.DMA,
        ],
        compiler_params=pltpu.CompilerParams(
            use_tc_tiling_on_sc=True,
            needs_layout_passes=False,
            disable_bounds_checks=True,
        ),
    )
    def k3(g_hbm, out_hbm, in_v, out_v, isem0, isem1, osem0, osem1):
        wid = lax.axis_index("s") * NC + lax.axis_index("c")
        base = wid * per_w
        lane = lax.iota(jnp.int32, 16)
        rpat = lax.shift_right_logical(lane, 2)
        cpat = lax.shift_left(lane & 3, 5)
        isems = (isem0, isem1)
        osems = (osem0, osem1)

        def locate(c):
            l = c // (_B // CB)
            blk = c % (_B // CB)
            b0 = pl.multiple_of(blk * CB, CB)
            g0 = pl.multiple_of(l * (_B // 4) + blk * (CB // 4), CB // 4)
            return l, b0, g0

        def start_in(c, b):
            _, _, g0 = locate(c)
            pltpu.async_copy(g_hbm.at[pl.ds(g0, CB // 4)], in_v.at[b],
                             isems[b])

        def wait_in(b):
            pltpu.make_async_copy(g_hbm.at[pl.ds(0, CB // 4)], in_v.at[b],
                                  isems[b]).wait()

        def start_out(c, b):
            l, b0, _ = locate(c)
            pltpu.async_copy(out_v.at[b], out_hbm.at[l, :, pl.ds(b0, CB)],
                             osems[b])

        def wait_out(b):
            pltpu.make_async_copy(out_v.at[b], out_hbm.at[0, :, pl.ds(0, CB)],
                                  osems[b]).wait()

        def transpose(ib, ob):
            def floop(f, _):
                cv = cpat + f
                @plsc.parallel_loop(0, CB // 16, unroll=8)
                def _(k):
                    vals = plsc.load_gather(ib, [rpat + 4 * k, cv])
                    ob[f, pl.ds(16 * k, 16)] = vals
                return ()
            lax.fori_loop(0, _D, floop, ())

        start_in(base + 0, 0)
        start_in(base + 1, 1)

        def pair(j, _):
            c0 = base + 2 * j
            wait_in(0)
            @pl.when(j > 0)
            def _():
                wait_out(0)
            transpose(in_v.at[0], out_v.at[0])
            @pl.when(2 * j + 2 < per_w)
            def _():
                start_in(c0 + 2, 0)
            start_out(c0, 0)
            wait_in(1)
            @pl.when(j > 0)
            def _():
                wait_out(1)
            transpose(in_v.at[1], out_v.at[1])
            @pl.when(2 * j + 3 < per_w)
            def _():
                start_in(c0 + 3, 1)
            start_out(c0 + 1, 1)
            return ()

        lax.fori_loop(0, per_w // 2, pair, ())
        wait_out(0)
        wait_out(1)

    return k3


def kernel(indices, table):
    tabT = table.T
    tail = lax.slice(tabT, (0, 999936), (_D, _V))
    t128 = _make_k1()(tabT, tail)
    tbl = jnp.reshape(t128, (_V, _D))
    flat = jnp.reshape(jnp.transpose(indices), (_BT,))
    g = _make_k2()(flat, tbl)
    g128 = jnp.reshape(g, (_BT // 4, 128))
    o = _make_k3()(g128)
    return jnp.transpose(o, (2, 0, 1))
